# baseline (speedup 1.0000x reference)
import jax
import jax.numpy as jnp
from jax import lax
from jax.experimental import pallas as pl
from jax.experimental.pallas import tpu as pltpu

PLANE = 96
INTER = 96
HEADS = 4
OUTP = 96
TOPK = 5
F = INTER * HEADS
N = 1024
B = 8
BN_ = B * N
EPS = 1e-5
CNT = float(B * N)

_f32 = jnp.float32


def _dot(a, b, dims=((1,), (0,))):
    return lax.dot_general(a, b, (dims, ((), ())),
                           preferred_element_type=_f32)


def _headnorm(t):
    outs = []
    for h in range(HEADS):
        ch = t[h * INTER:(h + 1) * INTER, :]
        ss = jnp.sum(ch * ch, axis=0, keepdims=True)
        outs.append(ch / jnp.maximum(jnp.sqrt(ss), 1e-12))
    return outs


def _mono(x_ref, wkvq_ref, bkvq_ref, wp_ref, bp_ref, wg1_ref, bg1_ref,
          wg2_ref, bg2_ref, g1_ref, beta1_ref, g2_ref, beta2_ref,
          out_ref, z1_s):
    wkvq = wkvq_ref[...]
    bkvq = bkvq_ref[...]
    wg1 = wg1_ref[...]
    wpg = _dot(wg1, wp_ref[...])
    bpg = _dot(wg1, bp_ref[...]) + bg1_ref[...]

    for b in range(B):
        xf = x_ref[b]
        kvq = _dot(wkvq, xf) + bkvq
        kn = _headnorm(kvq[0:F, :])
        qn = _headnorm(kvq[2 * F:3 * F, :])
        q = jnp.concatenate(qn, axis=0)
        kvcat = jnp.concatenate(kn + [kvq[F:2 * F, :]], axis=0)

        ssx = jnp.sum(xf * xf, axis=0, keepdims=True)
        xn = xf / jnp.maximum(jnp.sqrt(ssx), 1e-12)
        sim = _dot(xn, xn, ((0,), (0,)))

        m1 = jnp.max(sim, axis=0, keepdims=True)
        m2 = jnp.max(jnp.where(sim == m1, -jnp.inf, sim),
                     axis=0, keepdims=True)
        m3 = jnp.max(jnp.where((sim == m1) | (sim == m2), -jnp.inf, sim),
                     axis=0, keepdims=True)
        m4 = jnp.max(jnp.where((sim == m1) | (sim == m2) | (sim == m3),
                               -jnp.inf, sim), axis=0, keepdims=True)
        t5 = jnp.max(jnp.where((sim == m1) | (sim == m2) | (sim == m3) |
                               (sim == m4), -jnp.inf, sim),
                     axis=0, keepdims=True)
        adjt = jnp.where(sim >= t5, 1.0, 0.0).astype(_f32)

        lap = _dot(kvcat, adjt) + kvcat
        kv = jnp.sum(lap[0:F, :] * lap[F:2 * F, :], axis=1,
                     keepdims=True) * (1.0 / 36.0)
        hydra = q * kv
        z1_s[:, b * N:(b + 1) * N] = _dot(wpg, hydra) + bpg

    z1 = z1_s[...]
    mean1 = jnp.sum(z1, axis=1, keepdims=True) / CNT
    var1 = jnp.sum(z1 * z1, axis=1, keepdims=True) / CNT - mean1 * mean1
    y = (z1 - mean1) * (lax.rsqrt(var1 + EPS) * g1_ref[...]) + beta1_ref[...]
    y = jnp.maximum(y, 0.0)
    z2 = _dot(wg2_ref[...], y) + bg2_ref[...]

    mean2 = jnp.sum(z2, axis=1, keepdims=True) / CNT
    var2 = jnp.sum(z2 * z2, axis=1, keepdims=True) / CNT - mean2 * mean2
    scale2 = lax.rsqrt(var2 + EPS) * g2_ref[...]
    shift2 = beta2_ref[...] - mean2 * scale2
    for b in range(B):
        out_ref[b] = jnp.maximum(z2[:, b * N:(b + 1) * N] * scale2 + shift2,
                                 0.0)


@jax.jit
def kernel(x, Wk, bk, Wq, bq, Wv, bv, Wp, bp, Wg1, bg1, Wg2, bg2,
           g1, beta1, g2, beta2):
    b, c, h, w = x.shape
    xr = x.reshape(b, c, h * w)
    col = lambda a: a.reshape(-1, 1)
    wkvq = jnp.concatenate([Wk, Wv, Wq], axis=0)
    bkvq = jnp.concatenate([bk, bv, bq]).reshape(-1, 1)

    out = pl.pallas_call(
        _mono,
        out_shape=jax.ShapeDtypeStruct((B, OUTP, N), _f32),
        scratch_shapes=[pltpu.VMEM((OUTP, BN_), _f32)],
    )(xr, wkvq, bkvq, Wp, col(bp), Wg1, col(bg1), Wg2, col(bg2),
      col(g1), col(beta1), col(g2), col(beta2))

    return out.reshape(b, OUTP, h, w)


# speedup vs baseline: 1.1609x; 1.1609x over previous
import jax
import jax.numpy as jnp
from jax import lax
from jax.experimental import pallas as pl
from jax.experimental.pallas import tpu as pltpu

PLANE = 96
INTER = 96
HEADS = 4
OUTP = 96
TOPK = 5
F = INTER * HEADS
N = 1024
B = 8
BN_ = B * N
EPS = 1e-5
CNT = float(B * N)

_f32 = jnp.float32


def _dot(a, b, dims=((1,), (0,))):
    return lax.dot_general(a, b, (dims, ((), ())),
                           preferred_element_type=_f32)


def _headnorm(t):
    outs = []
    for h in range(HEADS):
        ch = t[h * INTER:(h + 1) * INTER, :]
        ss = jnp.sum(ch * ch, axis=0, keepdims=True)
        outs.append(ch / jnp.maximum(jnp.sqrt(ss), 1e-12))
    return outs


def _mono(x_ref, wkvq_ref, bkvq_ref, wp_ref, bp_ref, wg1_ref, bg1_ref,
          wg2_ref, bg2_ref, g1_ref, beta1_ref, g2_ref, beta2_ref,
          out_ref, z1_s):
    wkvq = wkvq_ref[...]
    bkvq = bkvq_ref[...]
    wg1 = wg1_ref[...]
    wpg = _dot(wg1, wp_ref[...])
    bpg = _dot(wg1, bp_ref[...]) + bg1_ref[...]

    for b in range(B):
        xf = x_ref[b]
        kvq = _dot(wkvq, xf) + bkvq
        kn = _headnorm(kvq[0:F, :])
        qn = _headnorm(kvq[2 * F:3 * F, :])
        q = jnp.concatenate(qn, axis=0)
        kvcat = jnp.concatenate(kn + [kvq[F:2 * F, :]], axis=0)

        ssx = jnp.sum(xf * xf, axis=0, keepdims=True)
        xn = xf / jnp.maximum(jnp.sqrt(ssx), 1e-12)
        sim = _dot(xn, xn, ((0,), (0,)))

        s = sim
        for _ in range(TOPK - 1):
            m = jnp.max(s, axis=0, keepdims=True)
            s = jnp.where(s == m, -jnp.inf, s)
        t5 = jnp.max(s, axis=0, keepdims=True)
        adjt = jnp.where(sim >= t5, 1.0, 0.0).astype(_f32)

        lap = _dot(kvcat, adjt) + kvcat
        kv = jnp.sum(lap[0:F, :] * lap[F:2 * F, :], axis=1,
                     keepdims=True) * (1.0 / 36.0)
        hydra = q * kv
        z1_s[:, b * N:(b + 1) * N] = _dot(wpg, hydra) + bpg

    z1 = z1_s[...]
    mean1 = jnp.sum(z1, axis=1, keepdims=True) / CNT
    var1 = jnp.sum(z1 * z1, axis=1, keepdims=True) / CNT - mean1 * mean1
    y = (z1 - mean1) * (lax.rsqrt(var1 + EPS) * g1_ref[...]) + beta1_ref[...]
    y = jnp.maximum(y, 0.0)
    z2 = _dot(wg2_ref[...], y) + bg2_ref[...]

    mean2 = jnp.sum(z2, axis=1, keepdims=True) / CNT
    var2 = jnp.sum(z2 * z2, axis=1, keepdims=True) / CNT - mean2 * mean2
    scale2 = lax.rsqrt(var2 + EPS) * g2_ref[...]
    shift2 = beta2_ref[...] - mean2 * scale2
    for b in range(B):
        out_ref[b] = jnp.maximum(z2[:, b * N:(b + 1) * N] * scale2 + shift2,
                                 0.0)


@jax.jit
def kernel(x, Wk, bk, Wq, bq, Wv, bv, Wp, bp, Wg1, bg1, Wg2, bg2,
           g1, beta1, g2, beta2):
    b, c, h, w = x.shape
    xr = x.reshape(b, c, h * w)
    col = lambda a: a.reshape(-1, 1)
    wkvq = jnp.concatenate([Wk, Wv, Wq], axis=0)
    bkvq = jnp.concatenate([bk, bv, bq]).reshape(-1, 1)

    out = pl.pallas_call(
        _mono,
        out_shape=jax.ShapeDtypeStruct((B, OUTP, N), _f32),
        scratch_shapes=[pltpu.VMEM((OUTP, BN_), _f32)],
    )(xr, wkvq, bkvq, Wp, col(bp), Wg1, col(bg1), Wg2, col(bg2),
      col(g1), col(beta1), col(g2), col(beta2))

    return out.reshape(b, OUTP, h, w)


# speedup vs baseline: 1.2851x; 1.1070x over previous
import jax
import jax.numpy as jnp
from jax import lax
from jax.experimental import pallas as pl
from jax.experimental.pallas import tpu as pltpu

PLANE = 96
INTER = 96
HEADS = 4
OUTP = 96
TOPK = 5
F = INTER * HEADS
N = 1024
B = 8
BN_ = B * N
EPS = 1e-5
CNT = float(B * N)

_f32 = jnp.float32


def _dot(a, b, dims=((1,), (0,))):
    return lax.dot_general(a, b, (dims, ((), ())),
                           preferred_element_type=_f32)


def _headnorm(t):
    outs = []
    for h in range(HEADS):
        ch = t[h * INTER:(h + 1) * INTER, :]
        ss = jnp.sum(ch * ch, axis=0, keepdims=True)
        outs.append(ch / jnp.maximum(jnp.sqrt(ss), 1e-12))
    return outs


def _mono(x_ref, wkvq_ref, bkvq_ref, wp_ref, bp_ref, wg1_ref, bg1_ref,
          wg2_ref, bg2_ref, g1_ref, beta1_ref, g2_ref, beta2_ref,
          out_ref, z1_s):
    wkvq = wkvq_ref[...]
    bkvq = bkvq_ref[...]
    wg1 = wg1_ref[...]
    wpg = _dot(wg1, wp_ref[...])
    bpg = _dot(wg1, bp_ref[...]) + bg1_ref[...]

    for b in range(B):
        xf = x_ref[b]
        kvq = _dot(wkvq, xf) + bkvq
        kn = _headnorm(kvq[0:F, :])
        qn = _headnorm(kvq[2 * F:3 * F, :])
        q = jnp.concatenate(qn, axis=0)
        kvcat = jnp.concatenate(kn + [kvq[F:2 * F, :]], axis=0)

        ssx = jnp.sum(xf * xf, axis=0, keepdims=True)
        xn = xf / jnp.maximum(jnp.sqrt(ssx), 1e-12)
        sim = _dot(xn, xn, ((0,), (0,)))

        neg = jnp.full((8, N), -jnp.inf, _f32)
        t1 = t2 = t3 = t4 = t5 = neg
        sim3 = sim.reshape(N // 8, 8, N)
        for i in range(N // 8):
            xi = sim3[i]
            t1, xi = jnp.maximum(t1, xi), jnp.minimum(t1, xi)
            t2, xi = jnp.maximum(t2, xi), jnp.minimum(t2, xi)
            t3, xi = jnp.maximum(t3, xi), jnp.minimum(t3, xi)
            t4, xi = jnp.maximum(t4, xi), jnp.minimum(t4, xi)
            t5 = jnp.maximum(t5, xi)
        cand = jnp.concatenate([t1, t2, t3, t4, t5], axis=0)
        for _ in range(TOPK - 1):
            m = jnp.max(cand, axis=0, keepdims=True)
            cand = jnp.where(cand == m, -jnp.inf, cand)
        thr = jnp.max(cand, axis=0, keepdims=True)
        adjt = jnp.where(sim >= thr, 1.0, 0.0).astype(_f32)

        lap = _dot(kvcat, adjt) + kvcat
        kv = jnp.sum(lap[0:F, :] * lap[F:2 * F, :], axis=1,
                     keepdims=True) * (1.0 / 36.0)
        hydra = q * kv
        z1_s[:, b * N:(b + 1) * N] = _dot(wpg, hydra) + bpg

    z1 = z1_s[...]
    mean1 = jnp.sum(z1, axis=1, keepdims=True) / CNT
    var1 = jnp.sum(z1 * z1, axis=1, keepdims=True) / CNT - mean1 * mean1
    y = (z1 - mean1) * (lax.rsqrt(var1 + EPS) * g1_ref[...]) + beta1_ref[...]
    y = jnp.maximum(y, 0.0)
    z2 = _dot(wg2_ref[...], y) + bg2_ref[...]

    mean2 = jnp.sum(z2, axis=1, keepdims=True) / CNT
    var2 = jnp.sum(z2 * z2, axis=1, keepdims=True) / CNT - mean2 * mean2
    scale2 = lax.rsqrt(var2 + EPS) * g2_ref[...]
    shift2 = beta2_ref[...] - mean2 * scale2
    for b in range(B):
        out_ref[b] = jnp.maximum(z2[:, b * N:(b + 1) * N] * scale2 + shift2,
                                 0.0)


@jax.jit
def kernel(x, Wk, bk, Wq, bq, Wv, bv, Wp, bp, Wg1, bg1, Wg2, bg2,
           g1, beta1, g2, beta2):
    b, c, h, w = x.shape
    xr = x.reshape(b, c, h * w)
    col = lambda a: a.reshape(-1, 1)
    wkvq = jnp.concatenate([Wk, Wv, Wq], axis=0)
    bkvq = jnp.concatenate([bk, bv, bq]).reshape(-1, 1)

    out = pl.pallas_call(
        _mono,
        out_shape=jax.ShapeDtypeStruct((B, OUTP, N), _f32),
        scratch_shapes=[pltpu.VMEM((OUTP, BN_), _f32)],
    )(xr, wkvq, bkvq, Wp, col(bp), Wg1, col(bg1), Wg2, col(bg2),
      col(g1), col(beta1), col(g2), col(beta2))

    return out.reshape(b, OUTP, h, w)
